# baseline (speedup 1.0000x reference)
import functools

import jax
import jax.numpy as jnp
from jax import lax
from jax.experimental import pallas as pl
from jax.experimental.pallas import tpu as pltpu
from jax.experimental.pallas import tpu_sc as plsc

_NC = 2
_NS = 16
_NW = _NC * _NS
_CHUNK = 128


def _gather_body(nchunks, nbuf, uid_hbm, iid_hbm, ut_hbm, it_hbm,
                 uout_hbm, iout_hbm, idx_v, rows_v, sem_i, sem_g, sem_w):
    wid = lax.axis_index("s") * _NC + lax.axis_index("c")
    total = 2 * nchunks

    ld_u = pltpu.async_copy(uid_hbm.at[wid], idx_v.at[pl.ds(0, nchunks)],
                            sem_i)
    ld_i = pltpu.async_copy(iid_hbm.at[wid],
                            idx_v.at[pl.ds(nchunks, nchunks)], sem_i)
    ld_u.wait()
    ld_i.wait()

    def src_dst(p):
        if p < nchunks:
            return ut_hbm, uout_hbm, p
        return it_hbm, iout_hbm, p - nchunks

    ahead = nbuf - 2
    g = [None] * nbuf
    wb = [None] * nbuf

    def fire(p):
        slot = p % nbuf
        if wb[slot] is not None:
            wb[slot].wait()
            wb[slot] = None
        tab, _, _ = src_dst(p)
        g[slot] = pltpu.async_copy(tab.at[idx_v.at[p]], rows_v.at[slot],
                                   sem_g)

    for p in range(min(ahead + 1, total)):
        fire(p)
    for p in range(total):
        if p + ahead + 1 < total:
            fire(p + ahead + 1)
        slot = p % nbuf
        g[slot].wait()
        _, out_hbm, pp = src_dst(p)
        wb[slot] = pltpu.async_copy(rows_v.at[slot], out_hbm.at[wid, pp],
                                    sem_w)
    for w in wb:
        if w is not None:
            w.wait()


def _sc_gather(user_ids, item_ids, user_table, item_table):
    b = user_ids.shape[0]
    d = user_table.shape[1]
    b_per_w = b // _NW
    nchunks = b_per_w // _CHUNK

    mesh = plsc.VectorSubcoreMesh(
        core_axis_name="c", subcore_axis_name="s",
        num_cores=_NC, num_subcores=_NS)

    uid3 = user_ids.astype(jnp.int32).reshape(_NW, nchunks, _CHUNK)
    iid3 = item_ids.astype(jnp.int32).reshape(_NW, nchunks, _CHUNK)

    out_sds = jax.ShapeDtypeStruct((_NW, nchunks, _CHUNK, d), jnp.float32)
    nbuf = 7
    fn = pl.kernel(
        functools.partial(_gather_body, nchunks, nbuf),
        out_type=(out_sds, out_sds),
        mesh=mesh,
        scratch_types=[
            pltpu.VMEM((2 * nchunks, _CHUNK), jnp.int32),
            pltpu.VMEM((nbuf, _CHUNK, d), jnp.float32),
            pltpu.SemaphoreType.DMA,
            pltpu.SemaphoreType.DMA,
            pltpu.SemaphoreType.DMA,
        ],
    )
    u4, i4 = fn(uid3, iid3, user_table, item_table)
    return u4.reshape(b, d), i4.reshape(b, d)


def _dense_body(u_ref, i_ref, w1_ref, b1_ref, w2_ref, b2_ref, o_ref):
    d = u_ref.shape[1]
    u = u_ref[...]
    it = i_ref[...]
    mlp = jnp.dot(u, w1_ref[:d, :], preferred_element_type=jnp.float32)
    mlp += jnp.dot(it, w1_ref[d:, :], preferred_element_type=jnp.float32)
    mlp = jnp.maximum(mlp + b1_ref[...], 0.0)
    gmf = u * it
    s = jnp.sum(gmf * w2_ref[:, :d] + mlp * w2_ref[:, d:], axis=1)
    s += b2_ref[0, 0]
    o_ref[...] = 1.0 / (1.0 + jnp.exp(-s))


def _tc_dense(u_emb, i_emb, W1, b1, W2, b2, blk=2048, nrows=None):
    b, d = u_emb.shape
    if nrows is not None:
        b = nrows
    h = W1.shape[1]
    w2_row = W2.reshape(1, d + h)
    b1_2d = b1.reshape(1, h)
    b2_2d = b2.reshape(1, 1)
    out = pl.pallas_call(
        _dense_body,
        grid=(b // blk,),
        in_specs=[
            pl.BlockSpec((blk, d), lambda i: (i, 0)),
            pl.BlockSpec((blk, d), lambda i: (i, 0)),
            pl.BlockSpec((2 * d, h), lambda i: (0, 0)),
            pl.BlockSpec((1, h), lambda i: (0, 0)),
            pl.BlockSpec((1, d + h), lambda i: (0, 0)),
            pl.BlockSpec((1, 1), lambda i: (0, 0)),
        ],
        out_specs=pl.BlockSpec((blk,), lambda i: (i,)),
        out_shape=jax.ShapeDtypeStruct((b,), jnp.float32),
    )(u_emb, i_emb, W1, b1_2d, w2_row, b2_2d)
    return out


@jax.jit
def kernel(user_ids, item_ids, user_table, item_table, W1, b1, W2, b2):
    u_emb, i_emb = _sc_gather(user_ids, item_ids, user_table, item_table)
    return _tc_dense(u_emb, i_emb, W1, b1, W2, b2, blk=8192)


# speedup vs baseline: 1.0935x; 1.0935x over previous
import functools

import jax
import jax.numpy as jnp
from jax import lax
from jax.experimental import pallas as pl
from jax.experimental.pallas import tpu as pltpu
from jax.experimental.pallas import tpu_sc as plsc

_NC = 2
_NS = 16
_NW = _NC * _NS
_CHUNK = 128


def _gather_body(nchunks, nbuf, uid_hbm, iid_hbm, ut_hbm, it_hbm,
                 uout_hbm, iout_hbm, idx_v, rows_v, sem_i, sem_g, sem_w):
    wid = lax.axis_index("s") * _NC + lax.axis_index("c")
    total = 2 * nchunks

    ld_u = pltpu.async_copy(uid_hbm.at[wid], idx_v.at[pl.ds(0, nchunks)],
                            sem_i)
    ld_i = pltpu.async_copy(iid_hbm.at[wid],
                            idx_v.at[pl.ds(nchunks, nchunks)], sem_i)
    ld_u.wait()
    ld_i.wait()

    def src_dst(p):
        if p < nchunks:
            return ut_hbm, uout_hbm, p
        return it_hbm, iout_hbm, p - nchunks

    ahead = nbuf - 2
    g = [None] * nbuf
    wb = [None] * nbuf

    def fire(p):
        slot = p % nbuf
        if wb[slot] is not None:
            wb[slot].wait()
            wb[slot] = None
        tab, _, _ = src_dst(p)
        g[slot] = pltpu.async_copy(tab.at[idx_v.at[p]], rows_v.at[slot],
                                   sem_g)

    for p in range(min(ahead + 1, total)):
        fire(p)
    for p in range(total):
        if p + ahead + 1 < total:
            fire(p + ahead + 1)
        slot = p % nbuf
        g[slot].wait()
        _, out_hbm, pp = src_dst(p)
        wb[slot] = pltpu.async_copy(rows_v.at[slot], out_hbm.at[wid, pp],
                                    sem_w)
    for w in wb:
        if w is not None:
            w.wait()


def _sc_gather(user_ids, item_ids, user_table, item_table):
    b = user_ids.shape[0]
    d = user_table.shape[1]
    b_per_w = b // _NW
    nchunks = b_per_w // _CHUNK

    mesh = plsc.VectorSubcoreMesh(
        core_axis_name="c", subcore_axis_name="s",
        num_cores=_NC, num_subcores=_NS)

    uid3 = user_ids.astype(jnp.int32).reshape(_NW, nchunks, _CHUNK)
    iid3 = item_ids.astype(jnp.int32).reshape(_NW, nchunks, _CHUNK)

    out_sds = jax.ShapeDtypeStruct((_NW, nchunks, _CHUNK, d), jnp.float32)
    nbuf = 7
    fn = pl.kernel(
        functools.partial(_gather_body, nchunks, nbuf),
        out_type=(out_sds, out_sds),
        mesh=mesh,
        scratch_types=[
            pltpu.VMEM((2 * nchunks, _CHUNK), jnp.int32),
            pltpu.VMEM((nbuf, _CHUNK, d), jnp.float32),
            pltpu.SemaphoreType.DMA,
            pltpu.SemaphoreType.DMA,
            pltpu.SemaphoreType.DMA,
        ],
    )
    u4, i4 = fn(uid3, iid3, user_table, item_table)
    return u4.reshape(b, d), i4.reshape(b, d)


def _dense_body(u_ref, i_ref, w1t_ref, b1_ref, w2_ref, b2_ref, o_ref):
    d = u_ref.shape[1]
    ut = jnp.transpose(u_ref[...])
    it = jnp.transpose(i_ref[...])
    mlp = jnp.dot(w1t_ref[:, :d], ut, preferred_element_type=jnp.float32)
    mlp += jnp.dot(w1t_ref[:, d:], it, preferred_element_type=jnp.float32)
    mlp = jnp.maximum(mlp + b1_ref[...], 0.0)
    gmf = ut * it
    s = jnp.sum(gmf * w2_ref[:d, :], axis=0, keepdims=True)
    s += jnp.sum(mlp * w2_ref[d:, :], axis=0, keepdims=True)
    s += b2_ref[0, 0]
    o_ref[...] = 1.0 / (1.0 + jnp.exp(-s))


def _tc_dense(u_emb, i_emb, W1, b1, W2, b2, blk=2048, nrows=None):
    b, d = u_emb.shape
    if nrows is not None:
        b = nrows
    h = W1.shape[1]
    w1t = W1.T
    b1_2d = b1.reshape(h, 1)
    b2_2d = b2.reshape(1, 1)
    out = pl.pallas_call(
        _dense_body,
        grid=(b // blk,),
        in_specs=[
            pl.BlockSpec((blk, d), lambda i: (i, 0)),
            pl.BlockSpec((blk, d), lambda i: (i, 0)),
            pl.BlockSpec((h, 2 * d), lambda i: (0, 0)),
            pl.BlockSpec((h, 1), lambda i: (0, 0)),
            pl.BlockSpec((d + h, 1), lambda i: (0, 0)),
            pl.BlockSpec((1, 1), lambda i: (0, 0)),
        ],
        out_specs=pl.BlockSpec((1, blk), lambda i: (0, i)),
        out_shape=jax.ShapeDtypeStruct((1, b), jnp.float32),
    )(u_emb, i_emb, w1t, b1_2d, W2, b2_2d)
    return out.reshape(b)


@jax.jit
def kernel(user_ids, item_ids, user_table, item_table, W1, b1, W2, b2):
    u_emb, i_emb = _sc_gather(user_ids, item_ids, user_table, item_table)
    return _tc_dense(u_emb, i_emb, W1, b1, W2, b2, blk=8192)
